# Initial kernel scaffold; baseline (speedup 1.0000x reference)
import jax
import jax.numpy as jnp
from jax.experimental import pallas as pl


def kernel(x, edge_index, W1, b1, W2, b2):
    raise NotImplementedError("write your pallas kernel here")


# speedup vs baseline: 26.1829x; 26.1829x over previous
import functools

import jax
import jax.numpy as jnp
from jax import lax
from jax.experimental import pallas as pl
from jax.experimental.pallas import tpu as pltpu
from jax.experimental.pallas import tpu_sc as plsc

N = 10000
E = 320000
D = 128

NC = 2
NS = 16
NW = NC * NS
CHUNK = 128
CPT = 80
E_PAD = NW * CPT * CHUNK
N_CHUNKS = E_PAD // CHUNK
N_PAD = 10112
RPT = N_PAD // NS
DUMMY = N

N_SPARE = N_PAD - N


def _sc_degree_body(dst_hbm, ones_hbm, zero_hbm, out_hbm, dst_v, ones_v, acc, sem):
    cid = lax.axis_index("c")
    sid = lax.axis_index("s")
    wid = sid * NC + cid
    pltpu.sync_copy(dst_hbm.at[pl.ds(wid * CPT, CPT)], dst_v)
    pltpu.sync_copy(ones_hbm, ones_v)
    pltpu.sync_copy(zero_hbm, acc.at[pl.ds(sid * RPT, RPT)])
    plsc.subcore_barrier()

    def body(j, carry):
        pltpu.sync_copy(ones_v, acc.at[dst_v.at[j]], add=True)
        return carry

    lax.fori_loop(0, CPT, body, 0, unroll=False)
    plsc.subcore_barrier()
    pltpu.sync_copy(
        acc.at[pl.ds(sid * RPT, RPT)], out_hbm.at[cid, pl.ds(sid * RPT, RPT)]
    )


def _sc_scatter_body(u_hbm, src_hbm, dst_hbm, zero_hbm, out_hbm,
                     src_v, dst_v, rows_v, acc, sem0, sem1):
    cid = lax.axis_index("c")
    sid = lax.axis_index("s")
    wid = sid * NC + cid
    sems = (sem0, sem1)
    half = CPT // 2

    pltpu.sync_copy(zero_hbm, acc.at[pl.ds(sid * RPT, RPT)])
    pltpu.sync_copy(dst_hbm.at[pl.ds(wid * CPT, CPT)], dst_v)
    plsc.subcore_barrier()

    for p in range(2):
        pltpu.sync_copy(
            src_hbm.at[pl.ds(wid * CPT + p * half, half)], src_v
        )
        pltpu.async_copy(u_hbm.at[src_v.at[0]], rows_v.at[0], sems[0])

        def body(j, carry):
            for b in range(2):
                k = 2 * j + b

                @pl.when(k + 1 < half)
                def _():
                    pltpu.async_copy(
                        u_hbm.at[src_v.at[k + 1]], rows_v.at[1 - b],
                        sems[1 - b]
                    )

                pltpu.make_async_copy(
                    u_hbm.at[src_v.at[k]], rows_v.at[b], sems[b]
                ).wait()
                pltpu.sync_copy(
                    rows_v.at[b], acc.at[dst_v.at[p * half + k]], add=True
                )
            return carry

        lax.fori_loop(0, half // 2, body, 0, unroll=False)

    plsc.subcore_barrier()
    pltpu.sync_copy(
        acc.at[pl.ds(sid * RPT, RPT)], out_hbm.at[cid, pl.ds(sid * RPT, RPT)]
    )


@functools.cache
def _sc_kernels():
    mesh = plsc.VectorSubcoreMesh(
        core_axis_name="c", subcore_axis_name="s", num_cores=NC, num_subcores=NS
    )
    sc_degree = pl.kernel(
        _sc_degree_body,
        mesh=mesh,
        out_type=jax.ShapeDtypeStruct((NC, N_PAD, D), jnp.float32),
        scratch_types=[
            pltpu.VMEM((CPT, CHUNK), jnp.int32),
            pltpu.VMEM((CHUNK, D), jnp.float32),
            pltpu.VMEM_SHARED((N_PAD, D), jnp.float32),
            pltpu.SemaphoreType.DMA,
        ],
    )
    sc_scatter = pl.kernel(
        _sc_scatter_body,
        mesh=mesh,
        out_type=jax.ShapeDtypeStruct((NC, N_PAD, D), jnp.float32),
        scratch_types=[
            pltpu.VMEM((CPT // 2, CHUNK), jnp.int32),
            pltpu.VMEM((CPT, CHUNK), jnp.int32),
            pltpu.VMEM((2, CHUNK, D), jnp.float32),
            pltpu.VMEM_SHARED((N_PAD, D), jnp.float32),
            pltpu.SemaphoreType.DMA,
            pltpu.SemaphoreType.DMA,
        ],
    )
    return sc_degree, sc_scatter


R = 1000
GRID = N // R


def _dinv(degp_ref):
    deg = degp_ref[0, :, 0:1] + degp_ref[1, :, 0:1] + 1.0
    return lax.rsqrt(deg)


def _t1_body(x_ref, w_ref, degp_ref, u_ref):
    h = jnp.dot(x_ref[...], w_ref[...],
                preferred_element_type=jnp.float32,
                precision=lax.Precision.HIGHEST)
    u_ref[...] = _dinv(degp_ref) * h


def _t2_body(s_ref, u_ref, degp_ref, w_ref, b_ref, o_ref):
    dinv = _dinv(degp_ref)
    z = jnp.maximum(dinv * (s_ref[0] + s_ref[1] + u_ref[...]) + b_ref[...], 0.0)
    h = jnp.dot(z, w_ref[...],
                preferred_element_type=jnp.float32,
                precision=lax.Precision.HIGHEST)
    o_ref[...] = dinv * h


def _t3_body(s_ref, u_ref, degp_ref, b_ref, o_ref):
    dinv = _dinv(degp_ref)
    o_ref[...] = dinv * (s_ref[0] + s_ref[1] + u_ref[...]) + b_ref[...]


_spec_rows = pl.BlockSpec((R, D), lambda i: (i, 0))
_spec_w = pl.BlockSpec((D, D), lambda i: (0, 0))
_spec_b = pl.BlockSpec((1, D), lambda i: (0, 0))
_spec_deg = pl.BlockSpec((2, R, D), lambda i: (0, i, 0))
_spec_s = pl.BlockSpec((2, R, D), lambda i: (0, i, 0))

_t1 = pl.pallas_call(
    _t1_body, grid=(GRID,),
    in_specs=[_spec_rows, _spec_w, _spec_deg],
    out_specs=_spec_rows,
    out_shape=jax.ShapeDtypeStruct((N, D), jnp.float32),
)
_t2 = pl.pallas_call(
    _t2_body, grid=(GRID,),
    in_specs=[_spec_s, _spec_rows, _spec_deg, _spec_w, _spec_b],
    out_specs=_spec_rows,
    out_shape=jax.ShapeDtypeStruct((N, D), jnp.float32),
)
_t3 = pl.pallas_call(
    _t3_body, grid=(GRID,),
    in_specs=[_spec_s, _spec_rows, _spec_deg, _spec_b],
    out_specs=_spec_rows,
    out_shape=jax.ShapeDtypeStruct((N, D), jnp.float32),
)


def kernel(x, edge_index, W1, b1, W2, b2):
    src = edge_index[0].astype(jnp.int32)
    dst = edge_index[1].astype(jnp.int32)
    pad = E_PAD - E
    pad_iota = jnp.arange(pad, dtype=jnp.int32)
    src = jnp.concatenate([src, pad_iota % N])
    dst = jnp.concatenate([dst, DUMMY + pad_iota % N_SPARE])
    src3 = src.reshape(N_CHUNKS, CHUNK)
    dst3 = dst.reshape(N_CHUNKS, CHUNK)

    ones_rows = jnp.ones((CHUNK, D), jnp.float32)
    zrow = jnp.zeros((RPT, D), jnp.float32)

    sc_degree, sc_scatter = _sc_kernels()
    degp = sc_degree(dst3, ones_rows, zrow)
    u1 = _t1(x, W1, degp)
    s1 = sc_scatter(u1, src3, dst3, zrow)
    u2 = _t2(s1, u1, degp, W2, b1.reshape(1, D))
    s2 = sc_scatter(u2, src3, dst3, zrow)
    out = _t3(s2, u2, degp, b2.reshape(1, D))
    return out
